# baseline (speedup 1.0000x reference)
import functools
import jax
import jax.numpy as jnp
from jax import lax
from jax.experimental import pallas as pl
from jax.experimental.pallas import tpu as pltpu
from jax.experimental.pallas import tpu_sc as plsc
from jax.experimental.layout import Format, Layout

B, L = 8, 448
SEQ_EMB = 32
RES_DIM = 128
PAIR_DIM = 64
MAX_REL = 32
NREL = 2 * MAX_REL + 1
NUM_EMB = 5

_HI = jax.lax.Precision.HIGHEST

NC, NS = 2, 16
NW = NC * NS
ROWS = B * L
RPW = ROWS // NW
JU = 4


def _prep_body(seq_ref, dih_ref, ent_ref, acc_ref, con_ref, emb_ref, pe_ref,
               rel_emb_ref, Wr_ref, br_ref, Wp_ref, bp_ref,
               res_out, relproj_out):
    seq = seq_ref[...]
    onehot = (seq[..., None] ==
              jax.lax.broadcasted_iota(jnp.int32, (B, L, NUM_EMB), 2)
              ).astype(jnp.float32)
    M = jax.lax.dot_general(emb_ref[...], Wr_ref[0:SEQ_EMB, :],
                            (((1,), (0,)), ((), ())), precision=_HI)
    res = jax.lax.dot_general(onehot.reshape(B * L, NUM_EMB), M,
                              (((1,), (0,)), ((), ())), precision=_HI)
    res = res + jax.lax.dot_general(
        dih_ref[...].reshape(B * L, 4), Wr_ref[SEQ_EMB:SEQ_EMB + 4, :],
        (((1,), (0,)), ((), ())), precision=_HI)
    res = res.reshape(B, L, RES_DIM)
    res = res + ent_ref[...][..., None] * Wr_ref[SEQ_EMB + 4, :][None, None, :]
    res = res + acc_ref[...][..., None] * Wr_ref[SEQ_EMB + 5, :][None, None, :]
    res = res + con_ref[...][..., None] * Wr_ref[SEQ_EMB + 6, :][None, None, :]
    res = res + br_ref[...][None, None, :]
    res = res + pe_ref[0, :L, :][None]
    res_out[...] = res
    relproj_out[...] = jax.lax.dot_general(
        rel_emb_ref[...], Wp_ref[2:, :], (((1,), (0,)), ((), ())),
        precision=_HI) + bp_ref[...][None, :]


_GDN = lax.GatherDimensionNumbers(offset_dims=(), collapsed_slice_dims=(0,),
                                  start_index_map=(0,))


def _bcast(ch, u):
    return lax.gather(ch, jnp.full((16, 1), u, jnp.int32), _GDN, (1,),
                      mode=lax.GatherScatterMode.PROMISE_IN_BOUNDS)


def _sc_pair_body(relproj_hbm, w01_hbm, p_hbm, c_hbm, out_hbm,
                  relv, wv, pv, cv, ov,
                  psem0, psem1, csem0, csem1, osem0, osem1):
    wid = lax.axis_index("s") * NC + lax.axis_index("c")
    base_row = wid * RPW
    pltpu.sync_copy(relproj_hbm, relv)
    pltpu.sync_copy(w01_hbm, wv)
    w0 = [wv[0, pl.ds(d * 16, 16)] for d in range(4)]
    w1 = [wv[1, pl.ds(d * 16, 16)] for d in range(4)]

    def row_bi(r):
        row = base_row + r
        b = row // L
        return b, row - b * L

    for s, (ps, cs) in ((0, (psem0, csem0)), (1, (psem1, csem1))):
        b, i = row_bi(s)
        pltpu.make_async_copy(p_hbm.at[b, i], pv.at[s], ps).start()
        pltpu.make_async_copy(c_hbm.at[b, i], cv.at[s], cs).start()

    def group_body(g, _):
        for s, (ps, cs, osm) in ((0, (psem0, csem0, osem0)),
                                 (1, (psem1, csem1, osem1))):
            r = g * 2 + s
            b, i = row_bi(r)
            pltpu.make_async_copy(p_hbm.at[b, i], pv.at[s], ps).wait()
            pltpu.make_async_copy(c_hbm.at[b, i], cv.at[s], cs).wait()
            @pl.when(g > 0)
            def _():
                pb_, pi_ = row_bi(r - 2)
                pltpu.make_async_copy(ov.at[pl.ds(s * L, L)],
                                      out_hbm.at[pb_, pi_], osm).wait()

            def j_body(jg, _):
                jbase = jg * 16
                chp = pv[s, pl.ds(jbase, 16)]
                chc = cv[s, pl.ds(jbase, 16)]
                klo = jnp.clip(jbase - i + MAX_REL, 0, 2 * MAX_REL)
                khi = jnp.clip(jbase + 15 - i + MAX_REL, 0, 2 * MAX_REL)

                @pl.when(klo == khi)
                def _():
                    rels = [relv[klo, pl.ds(d * 16, 16)] for d in range(4)]
                    for u in range(16):
                        pb = _bcast(chp, u)
                        cb = _bcast(chc, u)
                        row = s * L + jbase + u
                        for d in range(4):
                            ov[row, pl.ds(d * 16, 16)] = (
                                pb * w0[d] + cb * w1[d] + rels[d])

                @pl.when(klo != khi)
                def _():
                    for u in range(16):
                        pb = _bcast(chp, u)
                        cb = _bcast(chc, u)
                        k = jnp.clip(jbase + u - i + MAX_REL, 0, 2 * MAX_REL)
                        row = s * L + jbase + u
                        for d in range(4):
                            ov[row, pl.ds(d * 16, 16)] = (
                                pb * w0[d] + cb * w1[d]
                                + relv[k, pl.ds(d * 16, 16)])
                return 0

            lax.fori_loop(0, L // 16, j_body, 0, unroll=False)
            pltpu.make_async_copy(ov.at[pl.ds(s * L, L)],
                                  out_hbm.at[b, i], osm).start()

            @pl.when(r + 2 < RPW)
            def _():
                nb, ni = row_bi(r + 2)
                pltpu.make_async_copy(p_hbm.at[nb, ni], pv.at[s], ps).start()
                pltpu.make_async_copy(c_hbm.at[nb, ni], cv.at[s], cs).start()
        return 0

    lax.fori_loop(0, RPW // 2, group_body, 0, unroll=False)
    for s, osm in ((0, osem0), (1, osem1)):
        b, i = row_bi(RPW - 2 + s)
        pltpu.make_async_copy(ov.at[pl.ds(s * L, L)],
                              out_hbm.at[b, i], osm).wait()


@functools.lru_cache(maxsize=1)
def _sc_pair():
  return pl.kernel(
    _sc_pair_body,
    out_type=jax.ShapeDtypeStruct((B, L, L, PAIR_DIM), jnp.float32),
    mesh=plsc.VectorSubcoreMesh(core_axis_name="c", subcore_axis_name="s",
                                num_cores=NC, num_subcores=NS),
    scratch_types=[
        pltpu.VMEM((NREL, PAIR_DIM), jnp.float32),
        pltpu.VMEM((2, PAIR_DIM), jnp.float32),
        pltpu.VMEM((2, L), jnp.float32),
        pltpu.VMEM((2, L), jnp.float32),
        pltpu.VMEM((2 * L, PAIR_DIM), jnp.float32),
        pltpu.SemaphoreType.DMA,
        pltpu.SemaphoreType.DMA,
        pltpu.SemaphoreType.DMA,
        pltpu.SemaphoreType.DMA,
        pltpu.SemaphoreType.DMA,
        pltpu.SemaphoreType.DMA,
    ],
  )


def _impl(sequence_int, dihedral_features, pairing_probs, positional_entropy,
          coupling_matrix, accessibility, conservation, emb_table, pe,
          rel_emb, W_res, b_res, W_pair, b_pair):
    res, relproj = pl.pallas_call(
        _prep_body,
        out_shape=(
            jax.ShapeDtypeStruct((B, L, RES_DIM), jnp.float32),
            jax.ShapeDtypeStruct((NREL, PAIR_DIM), jnp.float32),
        ),
    )(sequence_int.astype(jnp.int32), dihedral_features, positional_entropy,
      accessibility, conservation, emb_table, pe, rel_emb, W_res, b_res,
      W_pair, b_pair)

    pair = _sc_pair()(relproj, W_pair[0:2, :], pairing_probs, coupling_matrix)
    return res, pair


@functools.lru_cache(maxsize=1)
def _jit_impl():
    dev = jax.sharding.SingleDeviceSharding(jax.devices()[0])
    return jax.jit(
        _impl,
        out_shardings=(
            Format(Layout(major_to_minor=(0, 1, 2)), dev),
            Format(Layout(major_to_minor=(0, 1, 2, 3)), dev),
        ))


def kernel(sequence_int, mask, dihedral_features, pairing_probs,
           positional_entropy, coupling_matrix, accessibility, conservation,
           emb_table, pe, rel_emb, W_res, b_res, W_pair, b_pair):
    res, pair = _jit_impl()(sequence_int, dihedral_features, pairing_probs,
                      positional_entropy, coupling_matrix, accessibility,
                      conservation, emb_table, pe, rel_emb, W_res, b_res,
                      W_pair, b_pair)
    return res, pair, mask


# speedup vs baseline: 1.3670x; 1.3670x over previous
import functools
import jax
import jax.numpy as jnp
from jax import lax
from jax.experimental import pallas as pl
from jax.experimental.pallas import tpu as pltpu
from jax.experimental.pallas import tpu_sc as plsc

B, L = 8, 448
SEQ_EMB = 32
RES_DIM = 128
PAIR_DIM = 64
MAX_REL = 32
NREL = 2 * MAX_REL + 1
NUM_EMB = 5
TW = 144

_HI = jax.lax.Precision.HIGHEST

NC, NS = 2, 16
NW = NC * NS
ROWS = B * L
RPW = ROWS // NW
CB = 8


def _prep_body(seq_ref, dih_ref, ent_ref, acc_ref, con_ref, emb_ref, pe_ref,
               rel_emb_ref, Wr_ref, br_ref, Wp_ref, bp_ref,
               res_out, e2t_out):
    seq = seq_ref[...]
    onehot = (seq[..., None] ==
              jax.lax.broadcasted_iota(jnp.int32, (B, L, NUM_EMB), 2)
              ).astype(jnp.float32)
    M = jax.lax.dot_general(emb_ref[...], Wr_ref[0:SEQ_EMB, :],
                            (((1,), (0,)), ((), ())), precision=_HI)
    res = jax.lax.dot_general(onehot.reshape(B * L, NUM_EMB), M,
                              (((1,), (0,)), ((), ())), precision=_HI)
    res = res + jax.lax.dot_general(
        dih_ref[...].reshape(B * L, 4), Wr_ref[SEQ_EMB:SEQ_EMB + 4, :],
        (((1,), (0,)), ((), ())), precision=_HI)
    res = res.reshape(B, L, RES_DIM)
    res = res + ent_ref[...][..., None] * Wr_ref[SEQ_EMB + 4, :][None, None, :]
    res = res + acc_ref[...][..., None] * Wr_ref[SEQ_EMB + 5, :][None, None, :]
    res = res + con_ref[...][..., None] * Wr_ref[SEQ_EMB + 6, :][None, None, :]
    res = res + br_ref[...][None, None, :]
    res = res + pe_ref[0, :L, :][None]
    res_out[...] = res

    relproj = jax.lax.dot_general(
        rel_emb_ref[...], Wp_ref[2:, :], (((1,), (0,)), ((), ())),
        precision=_HI) + bp_ref[...][None, :]
    kk = jax.lax.broadcasted_iota(jnp.int32, (NREL, TW), 0)
    tt = jax.lax.broadcasted_iota(jnp.int32, (NREL, TW), 1)
    main = (tt < 128) & (jnp.clip(tt - 63, 0, 2 * MAX_REL) == kk)
    relx = ((tt == 130) & (kk == 0)) | ((tt == 131) & (kk == 2 * MAX_REL))
    sel_r = (main | relx).astype(jnp.float32)
    kk2 = jax.lax.broadcasted_iota(jnp.int32, (2, TW), 0)
    tt2 = jax.lax.broadcasted_iota(jnp.int32, (2, TW), 1)
    sel_w = (((kk2 == 0) & (tt2 == 128)) |
             ((kk2 == 1) & (tt2 == 129))).astype(jnp.float32)
    e2t = jax.lax.dot_general(relproj, sel_r, (((0,), (0,)), ((), ())),
                              precision=_HI)
    e2t = e2t + jax.lax.dot_general(Wp_ref[0:2, :], sel_w,
                                    (((0,), (0,)), ((), ())), precision=_HI)
    e2t_out[...] = e2t


_GDN = lax.GatherDimensionNumbers(offset_dims=(), collapsed_slice_dims=(0,),
                                  start_index_map=(0,))


def _bcast(ch, u):
    return lax.gather(ch, jnp.full((16, 1), u, jnp.int32), _GDN, (1,),
                      mode=lax.GatherScatterMode.PROMISE_IN_BOUNDS)


def _perm(ch, idxv):
    return lax.gather(ch, idxv[:, None], _GDN, (1,),
                      mode=lax.GatherScatterMode.PROMISE_IN_BOUNDS)


def _sc_pair_body(e2t_hbm, p_hbm, c_hbm, out_hbm,
                  e2t, pv, cv, ov,
                  psem0, psem1, csem0, csem1, osem0, osem1):
    wid = lax.axis_index("s") * NC + lax.axis_index("c")
    base_row = wid * RPW
    pltpu.sync_copy(e2t_hbm, e2t)
    lane = lax.iota(jnp.int32, 16)

    def row_bi(r):
        row = base_row + r
        b = row // L
        return b, row - b * L

    for s, (ps, cs) in ((0, (psem0, csem0)), (1, (psem1, csem1))):
        b, i = row_bi(s)
        pltpu.make_async_copy(p_hbm.at[b, i], pv.at[s], ps).start()
        pltpu.make_async_copy(c_hbm.at[b, i], cv.at[s], cs).start()

    def group_body(g, _):
        for s, (ps, cs, osm) in ((0, (psem0, csem0, osem0)),
                                 (1, (psem1, csem1, osem1))):
            r = g * 2 + s
            b, i = row_bi(r)
            pltpu.make_async_copy(p_hbm.at[b, i], pv.at[s], ps).wait()
            pltpu.make_async_copy(c_hbm.at[b, i], cv.at[s], cs).wait()
            @pl.when(g > 0)
            def _():
                pb_, pi_ = row_bi(r - 2)
                pltpu.make_async_copy(ov.at[s], out_hbm.at[pb_, pi_],
                                      osm).wait()

            glo = jnp.maximum((i - (MAX_REL - 1)) // 16, 0)
            ghi = jnp.minimum((i + (MAX_REL - 1)) // 16, L // 16 - 1)
            woff = 63 - i
            rot = woff % 16
            ashift = woff - rot
            idxv = (lane + rot) % 16
            lmask = lane < (16 - rot)

            for cb in range(PAIR_DIM // CB):
                aux = [None] * CB
                w0s = [None] * CB
                w1s = [None] * CB
                rel0s = [None] * CB
                rel64s = [None] * CB
                for cc in range(CB):
                    ch = cb * CB + cc
                    aux[cc] = e2t[ch, pl.ds(128, 16)]
                    w0s[cc] = _bcast(aux[cc], 0)
                    w1s[cc] = _bcast(aux[cc], 1)
                    rel0s[cc] = _bcast(aux[cc], 2)
                    rel64s[cc] = _bcast(aux[cc], 3)

                def mk_const(rels):
                    def body(jg, _):
                        jbase = pl.multiple_of(jg * 16, 16)
                        pch = pv[s, pl.ds(jbase, 16)]
                        cch = cv[s, pl.ds(jbase, 16)]
                        for cc in range(CB):
                            ch = cb * CB + cc
                            ov[s, ch, pl.ds(jbase, 16)] = (
                                pch * w0s[cc] + cch * w1s[cc] + rels[cc])
                        return 0
                    return body

                def band_body(jg, _):
                    jbase = pl.multiple_of(jg * 16, 16)
                    pch = pv[s, pl.ds(jbase, 16)]
                    cch = cv[s, pl.ds(jbase, 16)]
                    a = pl.multiple_of(jbase + ashift, 16)
                    for cc in range(CB):
                        ch = cb * CB + cc
                        c0 = e2t[ch, pl.ds(a, 16)]
                        c1 = e2t[ch, pl.ds(a + 16, 16)]
                        relt = jnp.where(lmask, _perm(c0, idxv),
                                         _perm(c1, idxv))
                        ov[s, ch, pl.ds(jbase, 16)] = (
                            pch * w0s[cc] + cch * w1s[cc] + relt)
                    return 0

                lax.fori_loop(0, glo, mk_const(rel0s), 0, unroll=False)
                lax.fori_loop(glo, ghi + 1, band_body, 0, unroll=False)
                lax.fori_loop(ghi + 1, L // 16, mk_const(rel64s), 0,
                              unroll=False)

            pltpu.make_async_copy(ov.at[s], out_hbm.at[b, i], osm).start()

            @pl.when(r + 2 < RPW)
            def _():
                nb, ni = row_bi(r + 2)
                pltpu.make_async_copy(p_hbm.at[nb, ni], pv.at[s], ps).start()
                pltpu.make_async_copy(c_hbm.at[nb, ni], cv.at[s], cs).start()
        return 0

    lax.fori_loop(0, RPW // 2, group_body, 0, unroll=False)
    for s, osm in ((0, osem0), (1, osem1)):
        b, i = row_bi(RPW - 2 + s)
        pltpu.make_async_copy(ov.at[s], out_hbm.at[b, i], osm).wait()


@functools.lru_cache(maxsize=1)
def _sc_pair():
  return pl.kernel(
    _sc_pair_body,
    out_type=jax.ShapeDtypeStruct((B, L, PAIR_DIM, L), jnp.float32),
    mesh=plsc.VectorSubcoreMesh(core_axis_name="c", subcore_axis_name="s",
                                num_cores=NC, num_subcores=NS),
    scratch_types=[
        pltpu.VMEM((PAIR_DIM, TW), jnp.float32),
        pltpu.VMEM((2, L), jnp.float32),
        pltpu.VMEM((2, L), jnp.float32),
        pltpu.VMEM((2, PAIR_DIM, L), jnp.float32),
        pltpu.SemaphoreType.DMA,
        pltpu.SemaphoreType.DMA,
        pltpu.SemaphoreType.DMA,
        pltpu.SemaphoreType.DMA,
        pltpu.SemaphoreType.DMA,
        pltpu.SemaphoreType.DMA,
    ],
  )


@jax.jit
def _impl(sequence_int, dihedral_features, pairing_probs, positional_entropy,
          coupling_matrix, accessibility, conservation, emb_table, pe,
          rel_emb, W_res, b_res, W_pair, b_pair):
    res, e2t = pl.pallas_call(
        _prep_body,
        out_shape=(
            jax.ShapeDtypeStruct((B, L, RES_DIM), jnp.float32),
            jax.ShapeDtypeStruct((PAIR_DIM, TW), jnp.float32),
        ),
    )(sequence_int.astype(jnp.int32), dihedral_features, positional_entropy,
      accessibility, conservation, emb_table, pe, rel_emb, W_res, b_res,
      W_pair, b_pair)

    pair_t = _sc_pair()(e2t, pairing_probs, coupling_matrix)
    return res, jnp.swapaxes(pair_t, 2, 3)


def kernel(sequence_int, mask, dihedral_features, pairing_probs,
           positional_entropy, coupling_matrix, accessibility, conservation,
           emb_table, pe, rel_emb, W_res, b_res, W_pair, b_pair):
    res, pair = _impl(sequence_int, dihedral_features, pairing_probs,
                      positional_entropy, coupling_matrix, accessibility,
                      conservation, emb_table, pe, rel_emb, W_res, b_res,
                      W_pair, b_pair)
    return res, pair, mask


# speedup vs baseline: 1.3738x; 1.0049x over previous
import functools
import jax
import jax.numpy as jnp
from jax import lax
from jax.experimental import pallas as pl
from jax.experimental.pallas import tpu as pltpu
from jax.experimental.pallas import tpu_sc as plsc

B, L = 8, 448
SEQ_EMB = 32
RES_DIM = 128
PAIR_DIM = 64
MAX_REL = 32
NREL = 2 * MAX_REL + 1
NUM_EMB = 5
TW = 144

_HI = jax.lax.Precision.HIGHEST

NC, NS = 2, 16
NW = NC * NS
ROWS = B * L
RPW = ROWS // NW
CB = 8


def _prep_body(seq_ref, dih_ref, ent_ref, acc_ref, con_ref, emb_ref, pe_ref,
               rel_emb_ref, Wr_ref, br_ref, Wp_ref, bp_ref,
               res_out, e2t_out):
    seq = seq_ref[...]
    onehot = (seq[..., None] ==
              jax.lax.broadcasted_iota(jnp.int32, (B, L, NUM_EMB), 2)
              ).astype(jnp.float32)
    M = jax.lax.dot_general(emb_ref[...], Wr_ref[0:SEQ_EMB, :],
                            (((1,), (0,)), ((), ())), precision=_HI)
    res = jax.lax.dot_general(onehot.reshape(B * L, NUM_EMB), M,
                              (((1,), (0,)), ((), ())), precision=_HI)
    res = res + jax.lax.dot_general(
        dih_ref[...].reshape(B * L, 4), Wr_ref[SEQ_EMB:SEQ_EMB + 4, :],
        (((1,), (0,)), ((), ())), precision=_HI)
    res = res.reshape(B, L, RES_DIM)
    res = res + ent_ref[...][..., None] * Wr_ref[SEQ_EMB + 4, :][None, None, :]
    res = res + acc_ref[...][..., None] * Wr_ref[SEQ_EMB + 5, :][None, None, :]
    res = res + con_ref[...][..., None] * Wr_ref[SEQ_EMB + 6, :][None, None, :]
    res = res + br_ref[...][None, None, :]
    res = res + pe_ref[0, :L, :][None]
    res_out[...] = res

    relproj = jax.lax.dot_general(
        rel_emb_ref[...], Wp_ref[2:, :], (((1,), (0,)), ((), ())),
        precision=_HI) + bp_ref[...][None, :]
    kk = jax.lax.broadcasted_iota(jnp.int32, (NREL, TW), 0)
    tt = jax.lax.broadcasted_iota(jnp.int32, (NREL, TW), 1)
    main = (tt < 128) & (jnp.clip(tt - 32, 0, 2 * MAX_REL) == kk)
    relx = ((tt == 130) & (kk == 0)) | ((tt == 131) & (kk == 2 * MAX_REL))
    sel_r = (main | relx).astype(jnp.float32)
    kk2 = jax.lax.broadcasted_iota(jnp.int32, (2, TW), 0)
    tt2 = jax.lax.broadcasted_iota(jnp.int32, (2, TW), 1)
    sel_w = (((kk2 == 0) & (tt2 == 128)) |
             ((kk2 == 1) & (tt2 == 129))).astype(jnp.float32)
    e2t = jax.lax.dot_general(relproj, sel_r, (((0,), (0,)), ((), ())),
                              precision=_HI)
    e2t = e2t + jax.lax.dot_general(Wp_ref[0:2, :], sel_w,
                                    (((0,), (0,)), ((), ())), precision=_HI)
    e2t_out[...] = e2t


_GDN = lax.GatherDimensionNumbers(offset_dims=(), collapsed_slice_dims=(0,),
                                  start_index_map=(0,))


def _bcast(ch, u):
    return lax.gather(ch, jnp.full((16, 1), u, jnp.int32), _GDN, (1,),
                      mode=lax.GatherScatterMode.PROMISE_IN_BOUNDS)


def _perm(ch, idxv):
    return lax.gather(ch, idxv[:, None], _GDN, (1,),
                      mode=lax.GatherScatterMode.PROMISE_IN_BOUNDS)


def _sc_pair_body(e2t_hbm, p_hbm, c_hbm, out_hbm,
                  e2t, pv, cv, ov,
                  psem0, psem1, csem0, csem1, osem0, osem1):
    wid = lax.axis_index("s") * NC + lax.axis_index("c")
    base_row = wid * RPW
    pltpu.sync_copy(e2t_hbm, e2t)
    lane = lax.iota(jnp.int32, 16)

    def row_bi(r):
        row = base_row + r
        b = row // L
        return b, row - b * L

    for s, (ps, cs) in ((0, (psem0, csem0)), (1, (psem1, csem1))):
        b, i = row_bi(s)
        pltpu.make_async_copy(p_hbm.at[b, i], pv.at[s], ps).start()
        pltpu.make_async_copy(c_hbm.at[b, i], cv.at[s], cs).start()

    def group_body(g, _):
        for s, (ps, cs, osm) in ((0, (psem0, csem0, osem0)),
                                 (1, (psem1, csem1, osem1))):
            r = g * 2 + s
            b, i = row_bi(r)
            pltpu.make_async_copy(p_hbm.at[b, i], pv.at[s], ps).wait()
            pltpu.make_async_copy(c_hbm.at[b, i], cv.at[s], cs).wait()
            @pl.when(g > 0)
            def _():
                pb_, pi_ = row_bi(r - 2)
                pltpu.make_async_copy(ov.at[s], out_hbm.at[pb_, pi_],
                                      osm).wait()

            glo = jnp.maximum((i - (MAX_REL - 1)) // 16, 0)
            ghi = jnp.minimum((i + (MAX_REL - 1)) // 16, L // 16 - 1)
            woff = 64 - i
            rot = woff % 16
            ashift = woff - rot
            idxv = (lane + rot) % 16
            lmask = lane < (16 - rot)

            for cb in range(PAIR_DIM // CB):
                aux = [None] * CB
                w0s = [None] * CB
                w1s = [None] * CB
                rel0s = [None] * CB
                rel64s = [None] * CB
                for cc in range(CB):
                    ch = cb * CB + cc
                    aux[cc] = e2t[ch, pl.ds(128, 16)]
                    w0s[cc] = _bcast(aux[cc], 0)
                    w1s[cc] = _bcast(aux[cc], 1)
                    rel0s[cc] = _bcast(aux[cc], 2)
                    rel64s[cc] = _bcast(aux[cc], 3)

                def mk_const(rels):
                    def body(jg, _):
                        jbase = pl.multiple_of(jg * 16, 16)
                        pch = pv[s, pl.ds(jbase, 16)]
                        cch = cv[s, pl.ds(jbase, 16)]
                        for cc in range(CB):
                            ch = cb * CB + cc
                            ov[s, ch, pl.ds(jbase, 16)] = (
                                pch * w0s[cc] + cch * w1s[cc] + rels[cc])
                        return 0
                    return body

                def band_body(jg, _):
                    jbase = pl.multiple_of(jg * 16, 16)
                    pch = pv[s, pl.ds(jbase, 16)]
                    cch = cv[s, pl.ds(jbase, 16)]
                    a = pl.multiple_of(jbase + ashift, 16)
                    for cc in range(CB):
                        ch = cb * CB + cc
                        c0 = e2t[ch, pl.ds(a, 16)]
                        c1 = e2t[ch, pl.ds(a + 16, 16)]
                        relt = jnp.where(lmask, _perm(c0, idxv),
                                         _perm(c1, idxv))
                        ov[s, ch, pl.ds(jbase, 16)] = (
                            pch * w0s[cc] + cch * w1s[cc] + relt)
                    return 0

                lax.fori_loop(0, glo, mk_const(rel0s), 0, unroll=False)
                lax.fori_loop(glo, ghi + 1, band_body, 0, unroll=False)
                lax.fori_loop(ghi + 1, L // 16, mk_const(rel64s), 0,
                              unroll=False)

            pltpu.make_async_copy(ov.at[s], out_hbm.at[b, i], osm).start()

            @pl.when(r + 2 < RPW)
            def _():
                nb, ni = row_bi(r + 2)
                pltpu.make_async_copy(p_hbm.at[nb, ni], pv.at[s], ps).start()
                pltpu.make_async_copy(c_hbm.at[nb, ni], cv.at[s], cs).start()
        return 0

    lax.fori_loop(0, RPW // 2, group_body, 0, unroll=False)
    for s, osm in ((0, osem0), (1, osem1)):
        b, i = row_bi(RPW - 2 + s)
        pltpu.make_async_copy(ov.at[s], out_hbm.at[b, i], osm).wait()


@functools.lru_cache(maxsize=1)
def _sc_pair():
  return pl.kernel(
    _sc_pair_body,
    out_type=jax.ShapeDtypeStruct((B, L, PAIR_DIM, L), jnp.float32),
    mesh=plsc.VectorSubcoreMesh(core_axis_name="c", subcore_axis_name="s",
                                num_cores=NC, num_subcores=NS),
    scratch_types=[
        pltpu.VMEM((PAIR_DIM, TW), jnp.float32),
        pltpu.VMEM((2, L), jnp.float32),
        pltpu.VMEM((2, L), jnp.float32),
        pltpu.VMEM((2, PAIR_DIM, L), jnp.float32),
        pltpu.SemaphoreType.DMA,
        pltpu.SemaphoreType.DMA,
        pltpu.SemaphoreType.DMA,
        pltpu.SemaphoreType.DMA,
        pltpu.SemaphoreType.DMA,
        pltpu.SemaphoreType.DMA,
    ],
  )


@jax.jit
def _impl(sequence_int, dihedral_features, pairing_probs, positional_entropy,
          coupling_matrix, accessibility, conservation, emb_table, pe,
          rel_emb, W_res, b_res, W_pair, b_pair):
    res, e2t = pl.pallas_call(
        _prep_body,
        out_shape=(
            jax.ShapeDtypeStruct((B, L, RES_DIM), jnp.float32),
            jax.ShapeDtypeStruct((PAIR_DIM, TW), jnp.float32),
        ),
    )(sequence_int.astype(jnp.int32), dihedral_features, positional_entropy,
      accessibility, conservation, emb_table, pe, rel_emb, W_res, b_res,
      W_pair, b_pair)

    pair_t = _sc_pair()(e2t, pairing_probs, coupling_matrix)
    return res, jnp.swapaxes(pair_t, 2, 3)


def kernel(sequence_int, mask, dihedral_features, pairing_probs,
           positional_entropy, coupling_matrix, accessibility, conservation,
           emb_table, pe, rel_emb, W_res, b_res, W_pair, b_pair):
    res, pair = _impl(sequence_int, dihedral_features, pairing_probs,
                      positional_entropy, coupling_matrix, accessibility,
                      conservation, emb_table, pe, rel_emb, W_res, b_res,
                      W_pair, b_pair)
    return res, pair, mask


# speedup vs baseline: 1.6942x; 1.2332x over previous
import functools
import jax
import jax.numpy as jnp
from jax import lax
from jax.experimental import pallas as pl
from jax.experimental.pallas import tpu as pltpu
from jax.experimental.pallas import tpu_sc as plsc

B, L = 8, 448
SEQ_EMB = 32
RES_DIM = 128
PAIR_DIM = 64
MAX_REL = 32
NREL = 2 * MAX_REL + 1
NUM_EMB = 5
TW = 144

_HI = jax.lax.Precision.HIGHEST

NC, NS = 2, 16
NW = NC * NS
ROWS = B * L
RPW = ROWS // NW
CB = 8


def _prep_body(seq_ref, dih_ref, ent_ref, acc_ref, con_ref, emb_ref, pe_ref,
               rel_emb_ref, Wr_ref, br_ref, Wp_ref, bp_ref,
               res_out, e2t_out):
    seq = seq_ref[...]
    onehot = (seq[..., None] ==
              jax.lax.broadcasted_iota(jnp.int32, (B, L, NUM_EMB), 2)
              ).astype(jnp.float32)
    M = jax.lax.dot_general(emb_ref[...], Wr_ref[0:SEQ_EMB, :],
                            (((1,), (0,)), ((), ())), precision=_HI)
    res = jax.lax.dot_general(onehot.reshape(B * L, NUM_EMB), M,
                              (((1,), (0,)), ((), ())), precision=_HI)
    res = res + jax.lax.dot_general(
        dih_ref[...].reshape(B * L, 4), Wr_ref[SEQ_EMB:SEQ_EMB + 4, :],
        (((1,), (0,)), ((), ())), precision=_HI)
    res = res.reshape(B, L, RES_DIM)
    res = res + ent_ref[...][..., None] * Wr_ref[SEQ_EMB + 4, :][None, None, :]
    res = res + acc_ref[...][..., None] * Wr_ref[SEQ_EMB + 5, :][None, None, :]
    res = res + con_ref[...][..., None] * Wr_ref[SEQ_EMB + 6, :][None, None, :]
    res = res + br_ref[...][None, None, :]
    res = res + pe_ref[0, :L, :][None]
    res_out[...] = res

    relproj = jax.lax.dot_general(
        rel_emb_ref[...], Wp_ref[2:, :], (((1,), (0,)), ((), ())),
        precision=_HI) + bp_ref[...][None, :]
    kk = jax.lax.broadcasted_iota(jnp.int32, (NREL, TW), 0)
    tt = jax.lax.broadcasted_iota(jnp.int32, (NREL, TW), 1)
    main = (tt < 128) & (jnp.clip(tt - 32, 0, 2 * MAX_REL) == kk)
    relx = ((tt == 130) & (kk == 0)) | ((tt == 131) & (kk == 2 * MAX_REL))
    sel_r = (main | relx).astype(jnp.float32)
    kk2 = jax.lax.broadcasted_iota(jnp.int32, (2, TW), 0)
    tt2 = jax.lax.broadcasted_iota(jnp.int32, (2, TW), 1)
    sel_w = (((kk2 == 0) & (tt2 == 128)) |
             ((kk2 == 1) & (tt2 == 129))).astype(jnp.float32)
    e2t = jax.lax.dot_general(relproj, sel_r, (((0,), (0,)), ((), ())),
                              precision=_HI)
    e2t = e2t + jax.lax.dot_general(Wp_ref[0:2, :], sel_w,
                                    (((0,), (0,)), ((), ())), precision=_HI)
    e2t_out[...] = e2t


_GDN = lax.GatherDimensionNumbers(offset_dims=(), collapsed_slice_dims=(0,),
                                  start_index_map=(0,))


def _bcast(ch, u):
    return lax.gather(ch, jnp.full((16, 1), u, jnp.int32), _GDN, (1,),
                      mode=lax.GatherScatterMode.PROMISE_IN_BOUNDS)


def _perm(ch, idxv):
    return lax.gather(ch, idxv[:, None], _GDN, (1,),
                      mode=lax.GatherScatterMode.PROMISE_IN_BOUNDS)


def _sc_pair_body(e2t_hbm, p_hbm, c_hbm, out_hbm,
                  e2t, pv, cv, ov,
                  psem0, psem1, csem0, csem1, osem0, osem1):
    wid = lax.axis_index("s") * NC + lax.axis_index("c")
    base_row = wid * RPW
    pltpu.sync_copy(e2t_hbm, e2t)
    lane = lax.iota(jnp.int32, 16)

    def row_bi(r):
        row = base_row + r
        b = row // L
        return b, row - b * L

    for s, (ps, cs) in ((0, (psem0, csem0)), (1, (psem1, csem1))):
        b, i = row_bi(s)
        pltpu.make_async_copy(p_hbm.at[b, i], pv.at[s], ps).start()
        pltpu.make_async_copy(c_hbm.at[b, i], cv.at[s], cs).start()

    def group_body(g, _):
        for s, (ps, cs, osm) in ((0, (psem0, csem0, osem0)),
                                 (1, (psem1, csem1, osem1))):
            r = g * 2 + s
            b, i = row_bi(r)
            pltpu.make_async_copy(p_hbm.at[b, i], pv.at[s], ps).wait()
            pltpu.make_async_copy(c_hbm.at[b, i], cv.at[s], cs).wait()
            @pl.when(g > 0)
            def _():
                pb_, pi_ = row_bi(r - 2)
                pltpu.make_async_copy(ov.at[s], out_hbm.at[pb_, pi_],
                                      osm).wait()

            glo = jnp.maximum((i - (MAX_REL - 1)) // 16, 0)
            ghi = jnp.minimum((i + (MAX_REL - 1)) // 16, L // 16 - 1)
            woff = 64 - i
            rot = woff % 16
            ashift = woff - rot
            idxv = (lane + rot) % 16
            lmask = lane < (16 - rot)

            for cb in range(PAIR_DIM // CB):
                aux = [None] * CB
                w0s = [None] * CB
                w1s = [None] * CB
                rel0s = [None] * CB
                rel64s = [None] * CB
                for cc in range(CB):
                    ch = cb * CB + cc
                    aux[cc] = e2t[ch, pl.ds(128, 16)]
                    w0s[cc] = _bcast(aux[cc], 0)
                    w1s[cc] = _bcast(aux[cc], 1)
                    rel0s[cc] = _bcast(aux[cc], 2)
                    rel64s[cc] = _bcast(aux[cc], 3)

                def mk_const(rels):
                    def body(jg):
                        jbase = pl.multiple_of(jg * 16, 16)
                        pch = pv[s, pl.ds(jbase, 16)]
                        cch = cv[s, pl.ds(jbase, 16)]
                        for cc in range(CB):
                            ch = cb * CB + cc
                            ov[s, ch, pl.ds(jbase, 16)] = (
                                pch * w0s[cc] + cch * w1s[cc] + rels[cc])
                    return body

                def band_body(jg):
                    jbase = pl.multiple_of(jg * 16, 16)
                    pch = pv[s, pl.ds(jbase, 16)]
                    cch = cv[s, pl.ds(jbase, 16)]
                    a = pl.multiple_of(jbase + ashift, 16)
                    for cc in range(CB):
                        ch = cb * CB + cc
                        c0 = e2t[ch, pl.ds(a, 16)]
                        c1 = e2t[ch, pl.ds(a + 16, 16)]
                        relt = jnp.where(lmask, _perm(c0, idxv),
                                         _perm(c1, idxv))
                        ov[s, ch, pl.ds(jbase, 16)] = (
                            pch * w0s[cc] + cch * w1s[cc] + relt)

                plsc.parallel_loop(0, glo)(mk_const(rel0s))
                plsc.parallel_loop(glo, ghi + 1)(band_body)
                plsc.parallel_loop(ghi + 1, L // 16)(mk_const(rel64s))

            pltpu.make_async_copy(ov.at[s], out_hbm.at[b, i], osm).start()

            @pl.when(r + 2 < RPW)
            def _():
                nb, ni = row_bi(r + 2)
                pltpu.make_async_copy(p_hbm.at[nb, ni], pv.at[s], ps).start()
                pltpu.make_async_copy(c_hbm.at[nb, ni], cv.at[s], cs).start()
        return 0

    lax.fori_loop(0, RPW // 2, group_body, 0, unroll=False)
    for s, osm in ((0, osem0), (1, osem1)):
        b, i = row_bi(RPW - 2 + s)
        pltpu.make_async_copy(ov.at[s], out_hbm.at[b, i], osm).wait()


@functools.lru_cache(maxsize=1)
def _sc_pair():
  return pl.kernel(
    _sc_pair_body,
    out_type=jax.ShapeDtypeStruct((B, L, PAIR_DIM, L), jnp.float32),
    mesh=plsc.VectorSubcoreMesh(core_axis_name="c", subcore_axis_name="s",
                                num_cores=NC, num_subcores=NS),
    scratch_types=[
        pltpu.VMEM((PAIR_DIM, TW), jnp.float32),
        pltpu.VMEM((2, L), jnp.float32),
        pltpu.VMEM((2, L), jnp.float32),
        pltpu.VMEM((2, PAIR_DIM, L), jnp.float32),
        pltpu.SemaphoreType.DMA,
        pltpu.SemaphoreType.DMA,
        pltpu.SemaphoreType.DMA,
        pltpu.SemaphoreType.DMA,
        pltpu.SemaphoreType.DMA,
        pltpu.SemaphoreType.DMA,
    ],
  )


@jax.jit
def _impl(sequence_int, dihedral_features, pairing_probs, positional_entropy,
          coupling_matrix, accessibility, conservation, emb_table, pe,
          rel_emb, W_res, b_res, W_pair, b_pair):
    res, e2t = pl.pallas_call(
        _prep_body,
        out_shape=(
            jax.ShapeDtypeStruct((B, L, RES_DIM), jnp.float32),
            jax.ShapeDtypeStruct((PAIR_DIM, TW), jnp.float32),
        ),
    )(sequence_int.astype(jnp.int32), dihedral_features, positional_entropy,
      accessibility, conservation, emb_table, pe, rel_emb, W_res, b_res,
      W_pair, b_pair)

    pair_t = _sc_pair()(e2t, pairing_probs, coupling_matrix)
    return res, jnp.swapaxes(pair_t, 2, 3)


def kernel(sequence_int, mask, dihedral_features, pairing_probs,
           positional_entropy, coupling_matrix, accessibility, conservation,
           emb_table, pe, rel_emb, W_res, b_res, W_pair, b_pair):
    res, pair = _impl(sequence_int, dihedral_features, pairing_probs,
                      positional_entropy, coupling_matrix, accessibility,
                      conservation, emb_table, pe, rel_emb, W_res, b_res,
                      W_pair, b_pair)
    return res, pair, mask
